# baseline (speedup 1.0000x reference)
import functools

import jax
import jax.numpy as jnp
from jax import lax
from jax.experimental import pallas as pl
from jax.experimental.pallas import tpu as pltpu
from jax.experimental.pallas import tpu_sc as plsc

N_LANES = 16

_SPLAT_DN = lax.GatherDimensionNumbers(
    offset_dims=(), collapsed_slice_dims=(0,), start_index_map=(0,))


def _lane_splat(v16, lane):
    idx = jnp.full((N_LANES, 1), lane, jnp.int32)
    return lax.gather(v16, idx, _SPLAT_DN, slice_sizes=(1,),
                      mode=lax.GatherScatterMode.PROMISE_IN_BOUNDS)


def _make_count(E, n, R, NC, NS):
    NW = NC * NS
    per_w = E // NW
    mesh = plsc.VectorSubcoreMesh(core_axis_name="c", subcore_axis_name="s")
    per_s = (n * R) // NS

    @functools.partial(
        pl.kernel,
        mesh=mesh,
        out_type=jax.ShapeDtypeStruct((NC * n * R,), jnp.float32),
        scratch_types=[
            pltpu.VMEM((per_w,), jnp.int32),
            pltpu.VMEM((per_w,), jnp.int32),
            pltpu.VMEM((per_w,), jnp.int32),
            pltpu.VMEM((per_w,), jnp.float32),
            pltpu.VMEM_SHARED((n * R,), jnp.float32),
        ],
    )
    def count_kernel(t_hbm, r_hbm, out_hbm, tv, rv, segv, onesv, cnt_sh):
        c = lax.axis_index("c")
        s = lax.axis_index("s")
        wid = c * NS + s
        base = wid * per_w

        def zfill(i, _):
            onesv[pl.ds(i * N_LANES, N_LANES)] = jnp.zeros((N_LANES,), jnp.float32)
            return 0

        lax.fori_loop(0, per_w // N_LANES, zfill, 0)
        pltpu.sync_copy(onesv.at[pl.ds(0, per_s)], cnt_sh.at[pl.ds(s * per_s, per_s)])

        pltpu.sync_copy(t_hbm.at[pl.ds(base, per_w)], tv)
        pltpu.sync_copy(r_hbm.at[pl.ds(base, per_w)], rv)

        def seg_body(i, _):
            sl = pl.ds(i * N_LANES, N_LANES)
            segv[sl] = tv[sl] * R + rv[sl]
            onesv[sl] = jnp.ones((N_LANES,), jnp.float32)
            return 0

        lax.fori_loop(0, per_w // N_LANES, seg_body, 0)

        plsc.subcore_barrier()
        pltpu.sync_copy(onesv, cnt_sh.at[segv], add=True)
        plsc.subcore_barrier()

        pltpu.sync_copy(cnt_sh.at[pl.ds(s * per_s, per_s)], onesv)
        pltpu.sync_copy(onesv,
                        out_hbm.at[pl.ds(c * (n * R) + s * per_s, per_s)])

    return count_kernel


def _make_agg(E, n, R, D, NC, NS, B, CH, MEGA):
    Dh = D // NC
    per_tile = E // NS
    nmega = per_tile // MEGA
    nchunk = MEGA // CH
    nbatch = MEGA // B
    nb2 = nbatch // 2
    mesh = plsc.VectorSubcoreMesh(core_axis_name="c", subcore_axis_name="s")
    rows_a = 8 * (n // (8 * NS))
    rows_last = n - (NS - 1) * rows_a

    def _row_chunks(total):
        out, off = [], 0
        while off < total:
            sz = min(B, total - off)
            out.append((off, sz))
            off += sz
        return out

    @functools.partial(
        pl.kernel,
        mesh=mesh,
        out_type=jax.ShapeDtypeStruct((NC * n, Dh), jnp.float32),
        compiler_params=pltpu.CompilerParams(use_tc_tiling_on_sc=False),
        scratch_types=[
            pltpu.VMEM((CH,), jnp.int32),
            pltpu.VMEM((CH,), jnp.int32),
            pltpu.VMEM((MEGA,), jnp.int32),
            pltpu.VMEM((MEGA,), jnp.int32),
            pltpu.VMEM((CH,), jnp.int32),
            pltpu.VMEM((MEGA,), jnp.float32),
            pltpu.VMEM((B,), jnp.int32),
            pltpu.VMEM((B,), jnp.int32),
            pltpu.VMEM((B,), jnp.int32),
            pltpu.VMEM((B,), jnp.int32),
            pltpu.VMEM((B, Dh // 2), jnp.int32),
            pltpu.VMEM((B, Dh // 2), jnp.int32),
            pltpu.VMEM((B, Dh), jnp.float32),
            pltpu.VMEM_SHARED((n, Dh), jnp.float32),
            pltpu.SemaphoreType.DMA,
            pltpu.SemaphoreType.DMA,
        ],
    )
    def agg_kernel(h_hbm, r_hbm, t_hbm, z_hbm, winv_hbm, out_hbm,
                   hv, rv, tv, gidxv, segv, wv, gb0, tb0, gb1, tb1,
                   zbuf0, zbuf1, zf32, agg_sh, sem_g0, sem_g1):
        c = lax.axis_index("c")
        s = lax.axis_index("s")

        def zero_body(e, _):
            for k in range(Dh // N_LANES):
                zf32[e, pl.ds(k * N_LANES, N_LANES)] = jnp.zeros(
                    (N_LANES,), jnp.float32)
            return 0

        lax.fori_loop(0, B, zero_body, 0)

        @pl.when(s < NS - 1)
        def _():
            for off, sz in _row_chunks(rows_a):
                pltpu.sync_copy(zf32.at[pl.ds(0, sz), :],
                                agg_sh.at[pl.ds(s * rows_a + off, sz), :])

        @pl.when(s == NS - 1)
        def _():
            for off, sz in _row_chunks(rows_last):
                pltpu.sync_copy(
                    zf32.at[pl.ds(0, sz), :],
                    agg_sh.at[pl.ds((NS - 1) * rows_a + off, sz), :])

        plsc.subcore_barrier()

        def fill_idx(q, gb, tb):
            def cp(j, _):
                sl = pl.ds(j * N_LANES, N_LANES)
                src = pl.ds(q * B + j * N_LANES, N_LANES)
                gb[sl] = gidxv[src]
                tb[sl] = tv[src]
                return 0

            lax.fori_loop(0, B // N_LANES, cp, 0)

        def scale(zbuf, q):
            def group_body(g, _):
                w16 = wv[pl.ds(q * B + g * N_LANES, N_LANES)]
                for l in range(N_LANES):
                    wspl = _lane_splat(w16, l)
                    row = g * N_LANES + l
                    for k in range(Dh // (2 * N_LANES)):
                        raw = zbuf[row, pl.ds(k * N_LANES, N_LANES)]
                        lo = lax.bitcast_convert_type(
                            jnp.left_shift(raw, 16), jnp.float32)
                        hi = lax.bitcast_convert_type(
                            jnp.bitwise_and(raw, jnp.int32(-65536)),
                            jnp.float32)
                        zf32[row, pl.ds(k * 2 * N_LANES, N_LANES)] = (
                            lo * wspl)
                        zf32[row, pl.ds(k * 2 * N_LANES + N_LANES,
                                        N_LANES)] = hi * wspl
                return 0

            lax.fori_loop(0, B // N_LANES, group_body, 0, unroll=2)

        def mega_body(m, _):
            tbase = s * per_tile + m * MEGA

            pltpu.sync_copy(t_hbm.at[pl.ds(tbase, MEGA)], tv)

            def chunk_body(mc, _):
                off = mc * CH
                pltpu.sync_copy(h_hbm.at[pl.ds(tbase + off, CH)], hv)
                pltpu.sync_copy(r_hbm.at[pl.ds(tbase + off, CH)], rv)

                def idx_body(i, _):
                    sl = pl.ds(i * N_LANES, N_LANES)
                    dst = pl.ds(off + i * N_LANES, N_LANES)
                    r16 = rv[sl]
                    gidxv[dst] = (r16 * n + hv[sl]) * NC + c
                    segv[sl] = tv[dst] * R + r16
                    return 0

                lax.fori_loop(0, CH // N_LANES, idx_body, 0)
                pltpu.sync_copy(winv_hbm.at[segv], wv.at[pl.ds(off, CH)])
                return 0

            lax.fori_loop(0, nchunk, chunk_body, 0)

            fill_idx(0, gb0, tb0)
            pltpu.async_copy(z_hbm.at[gb0], zbuf0, sem_g0)

            def pair_body(p, _):
                pltpu.make_async_copy(z_hbm.at[gb0], zbuf0, sem_g0).wait()
                fill_idx(2 * p + 1, gb1, tb1)
                pltpu.async_copy(z_hbm.at[gb1], zbuf1, sem_g1)
                scale(zbuf0, 2 * p)
                pltpu.sync_copy(zf32, agg_sh.at[tb0], add=True)

                pltpu.make_async_copy(z_hbm.at[gb1], zbuf1, sem_g1).wait()
                fill_idx(2 * p + 2, gb0, tb0)
                pltpu.async_copy(z_hbm.at[gb0], zbuf0, sem_g0)
                scale(zbuf1, 2 * p + 1)
                pltpu.sync_copy(zf32, agg_sh.at[tb1], add=True)
                return 0

            lax.fori_loop(0, nb2, pair_body, 0)
            pltpu.make_async_copy(z_hbm.at[gb0], zbuf0, sem_g0).wait()
            scale(zbuf0, nbatch - 1)
            pltpu.sync_copy(zf32, agg_sh.at[tb0], add=True)
            return 0

        lax.fori_loop(0, nmega, mega_body, 0)

        plsc.subcore_barrier()

        @pl.when(s < NS - 1)
        def _():
            for off, sz in _row_chunks(rows_a):
                row0 = s * rows_a + off
                pltpu.sync_copy(agg_sh.at[pl.ds(row0, sz), :],
                                zf32.at[pl.ds(0, sz), :])
                pltpu.sync_copy(zf32.at[pl.ds(0, sz), :],
                                out_hbm.at[pl.ds(c * n + row0, sz), :])

        @pl.when(s == NS - 1)
        def _():
            for off, sz in _row_chunks(rows_last):
                row0 = (NS - 1) * rows_a + off
                pltpu.sync_copy(agg_sh.at[pl.ds(row0, sz), :],
                                zf32.at[pl.ds(0, sz), :])
                pltpu.sync_copy(zf32.at[pl.ds(0, sz), :],
                                out_hbm.at[pl.ds(c * n + row0, sz), :])

    return agg_kernel


def _winv_body(cnt_ref, winv_ref):
    c = cnt_ref[0] + cnt_ref[1]
    winv_ref[...] = 1.0 / jnp.maximum(c, 1.0)


def _winv(cnt, n, R):
    rows = (n * R) // 128
    cnt3 = cnt.reshape(2, rows, 128)
    out = pl.pallas_call(
        _winv_body,
        out_shape=jax.ShapeDtypeStruct((rows, 128), jnp.float32),
    )(cnt3)
    return out.reshape(n * R)


def _zself_body(x_ref, w_ref, root_ref, b_ref, z_ref, self_ref):
    r = pl.program_id(1)
    x = x_ref[...]
    z = jnp.dot(x, w_ref[0], preferred_element_type=jnp.float32)
    nb, d = z.shape
    z_int = lax.bitcast_convert_type(z, jnp.int32)

    def _r16(v):
        return jnp.right_shift(
            v + 0x7FFF + (jnp.right_shift(v, 16) & 1), 16) & 0xFFFF

    words = []
    for k in range(d // 32):
        lo = _r16(z_int[:, k * 32:k * 32 + 16])
        hi = _r16(z_int[:, k * 32 + 16:k * 32 + 32])
        words.append(lo | jnp.left_shift(hi, 16))
    z_ref[0] = jnp.concatenate(words, axis=1)

    @pl.when(r == 0)
    def _():
        self_ref[...] = (
            jnp.dot(x, root_ref[...], preferred_element_type=jnp.float32)
            + b_ref[...]
        )


def _zself(x, W, root, b, NB):
    n, d_in = x.shape
    R, _, d_out = W.shape
    ni = n // NB
    z, self_ = pl.pallas_call(
        _zself_body,
        grid=(ni, R),
        in_specs=[
            pl.BlockSpec((NB, d_in), lambda i, r: (i, 0)),
            pl.BlockSpec((1, d_in, d_out), lambda i, r: (r, 0, 0)),
            pl.BlockSpec((d_in, d_out), lambda i, r: (0, 0)),
            pl.BlockSpec((1, d_out), lambda i, r: (0, 0)),
        ],
        out_specs=[
            pl.BlockSpec((1, NB, d_out // 2), lambda i, r: (r, i, 0)),
            pl.BlockSpec((NB, d_out), lambda i, r: (i, 0)),
        ],
        out_shape=[
            jax.ShapeDtypeStruct((R, n, d_out // 2), jnp.int32),
            jax.ShapeDtypeStruct((n, d_out), jnp.float32),
        ],
    )(x, W, root, b.reshape(1, d_out))
    return z.reshape(R * n, d_out // 2), self_


def _combine_body(agg_ref, self_ref, out_ref, *, relu):
    v = jnp.concatenate([agg_ref[0], agg_ref[1]], axis=1) + self_ref[...]
    out_ref[...] = jnp.maximum(v, 0.0) if relu else v


def _combine(agg, self_, NB, relu):
    _, n, dh = agg.shape
    d = 2 * dh
    ni = n // NB
    return pl.pallas_call(
        functools.partial(_combine_body, relu=relu),
        grid=(ni,),
        in_specs=[
            pl.BlockSpec((2, NB, dh), lambda i: (0, i, 0)),
            pl.BlockSpec((NB, d), lambda i: (i, 0)),
        ],
        out_specs=pl.BlockSpec((NB, d), lambda i: (i, 0)),
        out_shape=jax.ShapeDtypeStruct((n, d), jnp.float32),
    )(agg, self_)


def _final_body(agg_ref, self_ref, out_ref, acc_ref, *, ni, n):
    i = pl.program_id(0)

    @pl.when(i == 0)
    def _():
        acc_ref[...] = jnp.zeros_like(acc_ref)

    x = jnp.concatenate([agg_ref[0], agg_ref[1]], axis=1) + self_ref[...]
    nb, d = x.shape
    acc_ref[...] += jnp.sum(x.reshape(nb // 8, 8, d), axis=0)

    @pl.when(i == ni - 1)
    def _():
        out_ref[...] = jnp.sum(acc_ref[...], axis=0, keepdims=True) * (1.0 / n)


def _final(agg, self_, NB):
    _, n, dh = agg.shape
    d = 2 * dh
    ni = n // NB
    return pl.pallas_call(
        functools.partial(_final_body, ni=ni, n=n),
        grid=(ni,),
        in_specs=[
            pl.BlockSpec((2, NB, dh), lambda i: (0, i, 0)),
            pl.BlockSpec((NB, d), lambda i: (i, 0)),
        ],
        out_specs=pl.BlockSpec((1, d), lambda i: (0, 0)),
        out_shape=jax.ShapeDtypeStruct((1, d), jnp.float32),
        scratch_shapes=[pltpu.VMEM((8, d), jnp.float32)],
    )(agg, self_)


def kernel(h, r, t, emb, W1, root1, b1, W2, root2, b2):
    E = h.shape[0]
    n, d_in = emb.shape
    R = W1.shape[0]
    D = W1.shape[2]

    info = plsc.get_sparse_core_info()
    NC, NS = info.num_cores, info.num_subcores

    NB = 1000
    B = 400
    CH = 2000
    MEGA = 10000
    Dh = D // NC

    count_k = _make_count(E, n, R, NC, NS)
    agg_k = _make_agg(E, n, R, D, NC, NS, B, CH, MEGA)

    cnt = count_k(t, r).reshape(NC, n * R)
    winv = _winv(cnt, n, R)

    z1, self1 = _zself(emb, W1, root1, b1, NB)
    agg1 = agg_k(h, r, t, z1.reshape(R * n * NC, Dh // 2), winv).reshape(NC, n, Dh)
    x1 = _combine(agg1, self1, NB, relu=True)

    z2, self2 = _zself(x1, W2, root2, b2, NB)
    agg2 = agg_k(h, r, t, z2.reshape(R * n * NC, Dh // 2), winv).reshape(NC, n, Dh)
    return _final(agg2, self2, NB)


# speedup vs baseline: 1.7740x; 1.7740x over previous
import functools

import jax
import jax.numpy as jnp
from jax import lax
from jax.experimental import pallas as pl
from jax.experimental.pallas import tpu as pltpu
from jax.experimental.pallas import tpu_sc as plsc

N_LANES = 16

_SPLAT_DN = lax.GatherDimensionNumbers(
    offset_dims=(), collapsed_slice_dims=(0,), start_index_map=(0,))


def _lane_splat(v16, lane):
    idx = jnp.full((N_LANES, 1), lane, jnp.int32)
    return lax.gather(v16, idx, _SPLAT_DN, slice_sizes=(1,),
                      mode=lax.GatherScatterMode.PROMISE_IN_BOUNDS)


def _make_count(E, n, R, NC, NS):
    NW = NC * NS
    per_w = E // NW
    mesh = plsc.VectorSubcoreMesh(core_axis_name="c", subcore_axis_name="s")
    per_s = (n * R) // NS

    @functools.partial(
        pl.kernel,
        mesh=mesh,
        out_type=jax.ShapeDtypeStruct((NC * n * R,), jnp.float32),
        scratch_types=[
            pltpu.VMEM((per_w,), jnp.int32),
            pltpu.VMEM((per_w,), jnp.int32),
            pltpu.VMEM((per_w,), jnp.int32),
            pltpu.VMEM((per_w,), jnp.float32),
            pltpu.VMEM_SHARED((n * R,), jnp.float32),
        ],
    )
    def count_kernel(t_hbm, r_hbm, out_hbm, tv, rv, segv, onesv, cnt_sh):
        c = lax.axis_index("c")
        s = lax.axis_index("s")
        wid = c * NS + s
        base = wid * per_w

        def zfill(i, _):
            onesv[pl.ds(i * N_LANES, N_LANES)] = jnp.zeros((N_LANES,), jnp.float32)
            return 0

        lax.fori_loop(0, per_w // N_LANES, zfill, 0)
        pltpu.sync_copy(onesv.at[pl.ds(0, per_s)], cnt_sh.at[pl.ds(s * per_s, per_s)])

        pltpu.sync_copy(t_hbm.at[pl.ds(base, per_w)], tv)
        pltpu.sync_copy(r_hbm.at[pl.ds(base, per_w)], rv)

        def seg_body(i, _):
            sl = pl.ds(i * N_LANES, N_LANES)
            segv[sl] = tv[sl] * R + rv[sl]
            onesv[sl] = jnp.ones((N_LANES,), jnp.float32)
            return 0

        lax.fori_loop(0, per_w // N_LANES, seg_body, 0)

        plsc.subcore_barrier()
        pltpu.sync_copy(onesv, cnt_sh.at[segv], add=True)
        plsc.subcore_barrier()

        pltpu.sync_copy(cnt_sh.at[pl.ds(s * per_s, per_s)], onesv)
        pltpu.sync_copy(onesv,
                        out_hbm.at[pl.ds(c * (n * R) + s * per_s, per_s)])

    return count_kernel


def _make_agg(E, n, R, D, NC, NS, B, CH, MEGA):
    Dh = D // NC
    per_tile = E // NS
    nmega = per_tile // MEGA
    nchunk = MEGA // CH
    nbatch = MEGA // B
    nb2 = nbatch // 2
    mesh = plsc.VectorSubcoreMesh(core_axis_name="c", subcore_axis_name="s")
    rows_a = 8 * (n // (8 * NS))
    rows_last = n - (NS - 1) * rows_a

    def _row_chunks(total):
        out, off = [], 0
        while off < total:
            sz = min(B, total - off)
            out.append((off, sz))
            off += sz
        return out

    @functools.partial(
        pl.kernel,
        mesh=mesh,
        out_type=jax.ShapeDtypeStruct((NC * n, Dh), jnp.float32),
        compiler_params=pltpu.CompilerParams(use_tc_tiling_on_sc=False),
        scratch_types=[
            pltpu.VMEM((CH,), jnp.int32),
            pltpu.VMEM((CH,), jnp.int32),
            pltpu.VMEM((MEGA,), jnp.int32),
            pltpu.VMEM((MEGA,), jnp.int32),
            pltpu.VMEM((CH,), jnp.int32),
            pltpu.VMEM((MEGA,), jnp.float32),
            pltpu.VMEM((B,), jnp.int32),
            pltpu.VMEM((B,), jnp.int32),
            pltpu.VMEM((B,), jnp.int32),
            pltpu.VMEM((B,), jnp.int32),
            pltpu.VMEM((B, Dh), jnp.float32),
            pltpu.VMEM((B, Dh), jnp.float32),
            pltpu.VMEM_SHARED((n, Dh), jnp.float32),
            pltpu.SemaphoreType.DMA,
            pltpu.SemaphoreType.DMA,
        ],
    )
    def agg_kernel(h_hbm, r_hbm, t_hbm, z_hbm, winv_hbm, out_hbm,
                   hv, rv, tv, gidxv, segv, wv, gb0, tb0, gb1, tb1,
                   zbuf0, zbuf1, agg_sh, sem_g0, sem_g1):
        c = lax.axis_index("c")
        s = lax.axis_index("s")

        def zero_body(e, _):
            for k in range(Dh // N_LANES):
                zbuf0[e, pl.ds(k * N_LANES, N_LANES)] = jnp.zeros(
                    (N_LANES,), jnp.float32)
            return 0

        lax.fori_loop(0, B, zero_body, 0)

        @pl.when(s < NS - 1)
        def _():
            for off, sz in _row_chunks(rows_a):
                pltpu.sync_copy(zbuf0.at[pl.ds(0, sz), :],
                                agg_sh.at[pl.ds(s * rows_a + off, sz), :])

        @pl.when(s == NS - 1)
        def _():
            for off, sz in _row_chunks(rows_last):
                pltpu.sync_copy(
                    zbuf0.at[pl.ds(0, sz), :],
                    agg_sh.at[pl.ds((NS - 1) * rows_a + off, sz), :])

        plsc.subcore_barrier()

        def fill_idx(q, gb, tb):
            def cp(j, _):
                sl = pl.ds(j * N_LANES, N_LANES)
                src = pl.ds(q * B + j * N_LANES, N_LANES)
                gb[sl] = gidxv[src]
                tb[sl] = tv[src]
                return 0

            lax.fori_loop(0, B // N_LANES, cp, 0)

        def scale(zbuf, q):
            def group_body(g, _):
                w16 = wv[pl.ds(q * B + g * N_LANES, N_LANES)]
                for l in range(N_LANES):
                    wspl = _lane_splat(w16, l)
                    row = g * N_LANES + l
                    for k in range(Dh // N_LANES):
                        sl = pl.ds(k * N_LANES, N_LANES)
                        zbuf[row, sl] = zbuf[row, sl] * wspl
                return 0

            lax.fori_loop(0, B // N_LANES, group_body, 0, unroll=2)

        def mega_body(m, _):
            tbase = s * per_tile + m * MEGA

            pltpu.sync_copy(t_hbm.at[pl.ds(tbase, MEGA)], tv)

            def chunk_body(mc, _):
                off = mc * CH
                pltpu.sync_copy(h_hbm.at[pl.ds(tbase + off, CH)], hv)
                pltpu.sync_copy(r_hbm.at[pl.ds(tbase + off, CH)], rv)

                def idx_body(i, _):
                    sl = pl.ds(i * N_LANES, N_LANES)
                    dst = pl.ds(off + i * N_LANES, N_LANES)
                    r16 = rv[sl]
                    gidxv[dst] = (r16 * n + hv[sl]) * NC + c
                    segv[sl] = tv[dst] * R + r16
                    return 0

                lax.fori_loop(0, CH // N_LANES, idx_body, 0)
                pltpu.sync_copy(winv_hbm.at[segv], wv.at[pl.ds(off, CH)])
                return 0

            lax.fori_loop(0, nchunk, chunk_body, 0)

            fill_idx(0, gb0, tb0)
            pltpu.async_copy(z_hbm.at[gb0], zbuf0, sem_g0)

            def pair_body(p, _):
                pltpu.make_async_copy(z_hbm.at[gb0], zbuf0, sem_g0).wait()
                fill_idx(2 * p + 1, gb1, tb1)
                pltpu.async_copy(z_hbm.at[gb1], zbuf1, sem_g1)
                scale(zbuf0, 2 * p)
                pltpu.sync_copy(zbuf0, agg_sh.at[tb0], add=True)

                pltpu.make_async_copy(z_hbm.at[gb1], zbuf1, sem_g1).wait()
                fill_idx(2 * p + 2, gb0, tb0)
                pltpu.async_copy(z_hbm.at[gb0], zbuf0, sem_g0)
                scale(zbuf1, 2 * p + 1)
                pltpu.sync_copy(zbuf1, agg_sh.at[tb1], add=True)
                return 0

            lax.fori_loop(0, nb2, pair_body, 0)
            pltpu.make_async_copy(z_hbm.at[gb0], zbuf0, sem_g0).wait()
            scale(zbuf0, nbatch - 1)
            pltpu.sync_copy(zbuf0, agg_sh.at[tb0], add=True)
            return 0

        lax.fori_loop(0, nmega, mega_body, 0)

        plsc.subcore_barrier()

        @pl.when(s < NS - 1)
        def _():
            for off, sz in _row_chunks(rows_a):
                row0 = s * rows_a + off
                pltpu.sync_copy(agg_sh.at[pl.ds(row0, sz), :],
                                zbuf0.at[pl.ds(0, sz), :])
                pltpu.sync_copy(zbuf0.at[pl.ds(0, sz), :],
                                out_hbm.at[pl.ds(c * n + row0, sz), :])

        @pl.when(s == NS - 1)
        def _():
            for off, sz in _row_chunks(rows_last):
                row0 = (NS - 1) * rows_a + off
                pltpu.sync_copy(agg_sh.at[pl.ds(row0, sz), :],
                                zbuf0.at[pl.ds(0, sz), :])
                pltpu.sync_copy(zbuf0.at[pl.ds(0, sz), :],
                                out_hbm.at[pl.ds(c * n + row0, sz), :])

    return agg_kernel


def _winv_body(cnt_ref, winv_ref):
    c = cnt_ref[0] + cnt_ref[1]
    winv_ref[...] = 1.0 / jnp.maximum(c, 1.0)


def _winv(cnt, n, R):
    rows = (n * R) // 128
    cnt3 = cnt.reshape(2, rows, 128)
    out = pl.pallas_call(
        _winv_body,
        out_shape=jax.ShapeDtypeStruct((rows, 128), jnp.float32),
    )(cnt3)
    return out.reshape(n * R)


def _zself_body(x_ref, w_ref, root_ref, b_ref, z_ref, self_ref):
    r = pl.program_id(1)
    x = x_ref[...]
    z_ref[0] = jnp.dot(x, w_ref[0], preferred_element_type=jnp.float32)

    @pl.when(r == 0)
    def _():
        self_ref[...] = (
            jnp.dot(x, root_ref[...], preferred_element_type=jnp.float32)
            + b_ref[...]
        )


def _zself(x, W, root, b, NB):
    n, d_in = x.shape
    R, _, d_out = W.shape
    ni = n // NB
    z, self_ = pl.pallas_call(
        _zself_body,
        grid=(ni, R),
        in_specs=[
            pl.BlockSpec((NB, d_in), lambda i, r: (i, 0)),
            pl.BlockSpec((1, d_in, d_out), lambda i, r: (r, 0, 0)),
            pl.BlockSpec((d_in, d_out), lambda i, r: (0, 0)),
            pl.BlockSpec((1, d_out), lambda i, r: (0, 0)),
        ],
        out_specs=[
            pl.BlockSpec((1, NB, d_out), lambda i, r: (r, i, 0)),
            pl.BlockSpec((NB, d_out), lambda i, r: (i, 0)),
        ],
        out_shape=[
            jax.ShapeDtypeStruct((R, n, d_out), jnp.float32),
            jax.ShapeDtypeStruct((n, d_out), jnp.float32),
        ],
    )(x, W, root, b.reshape(1, d_out))
    return z.reshape(R * n, d_out), self_


def _combine_body(agg_ref, self_ref, out_ref, *, relu):
    v = jnp.concatenate([agg_ref[0], agg_ref[1]], axis=1) + self_ref[...]
    out_ref[...] = jnp.maximum(v, 0.0) if relu else v


def _combine(agg, self_, NB, relu):
    _, n, dh = agg.shape
    d = 2 * dh
    ni = n // NB
    return pl.pallas_call(
        functools.partial(_combine_body, relu=relu),
        grid=(ni,),
        in_specs=[
            pl.BlockSpec((2, NB, dh), lambda i: (0, i, 0)),
            pl.BlockSpec((NB, d), lambda i: (i, 0)),
        ],
        out_specs=pl.BlockSpec((NB, d), lambda i: (i, 0)),
        out_shape=jax.ShapeDtypeStruct((n, d), jnp.float32),
    )(agg, self_)


def _final_body(agg_ref, self_ref, out_ref, acc_ref, *, ni, n):
    i = pl.program_id(0)

    @pl.when(i == 0)
    def _():
        acc_ref[...] = jnp.zeros_like(acc_ref)

    x = jnp.concatenate([agg_ref[0], agg_ref[1]], axis=1) + self_ref[...]
    nb, d = x.shape
    acc_ref[...] += jnp.sum(x.reshape(nb // 8, 8, d), axis=0)

    @pl.when(i == ni - 1)
    def _():
        out_ref[...] = jnp.sum(acc_ref[...], axis=0, keepdims=True) * (1.0 / n)


def _final(agg, self_, NB):
    _, n, dh = agg.shape
    d = 2 * dh
    ni = n // NB
    return pl.pallas_call(
        functools.partial(_final_body, ni=ni, n=n),
        grid=(ni,),
        in_specs=[
            pl.BlockSpec((2, NB, dh), lambda i: (0, i, 0)),
            pl.BlockSpec((NB, d), lambda i: (i, 0)),
        ],
        out_specs=pl.BlockSpec((1, d), lambda i: (0, 0)),
        out_shape=jax.ShapeDtypeStruct((1, d), jnp.float32),
        scratch_shapes=[pltpu.VMEM((8, d), jnp.float32)],
    )(agg, self_)


def kernel(h, r, t, emb, W1, root1, b1, W2, root2, b2):
    E = h.shape[0]
    n, d_in = emb.shape
    R = W1.shape[0]
    D = W1.shape[2]

    info = plsc.get_sparse_core_info()
    NC, NS = info.num_cores, info.num_subcores

    NB = 1000
    B = 400
    CH = 2000
    MEGA = 10000
    Dh = D // NC

    count_k = _make_count(E, n, R, NC, NS)
    agg_k = _make_agg(E, n, R, D, NC, NS, B, CH, MEGA)

    cnt = count_k(t, r).reshape(NC, n * R)
    winv = _winv(cnt, n, R)

    z1, self1 = _zself(emb, W1, root1, b1, NB)
    agg1 = agg_k(h, r, t, z1.reshape(R * n * NC, Dh), winv).reshape(NC, n, Dh)
    x1 = _combine(agg1, self1, NB, relu=True)

    z2, self2 = _zself(x1, W2, root2, b2, NB)
    agg2 = agg_k(h, r, t, z2.reshape(R * n * NC, Dh), winv).reshape(NC, n, Dh)
    return _final(agg2, self2, NB)


# speedup vs baseline: 1.7828x; 1.0050x over previous
import functools

import jax
import jax.numpy as jnp
from jax import lax
from jax.experimental import pallas as pl
from jax.experimental.pallas import tpu as pltpu
from jax.experimental.pallas import tpu_sc as plsc

N_LANES = 16

_SPLAT_DN = lax.GatherDimensionNumbers(
    offset_dims=(), collapsed_slice_dims=(0,), start_index_map=(0,))


def _lane_splat(v16, lane):
    idx = jnp.full((N_LANES, 1), lane, jnp.int32)
    return lax.gather(v16, idx, _SPLAT_DN, slice_sizes=(1,),
                      mode=lax.GatherScatterMode.PROMISE_IN_BOUNDS)


def _make_count(E, n, R, NC, NS):
    NW = NC * NS
    per_w = E // NW
    mesh = plsc.VectorSubcoreMesh(core_axis_name="c", subcore_axis_name="s")
    per_s = (n * R) // NS

    @functools.partial(
        pl.kernel,
        mesh=mesh,
        out_type=jax.ShapeDtypeStruct((NC * n * R,), jnp.float32),
        scratch_types=[
            pltpu.VMEM((per_w,), jnp.int32),
            pltpu.VMEM((per_w,), jnp.int32),
            pltpu.VMEM((per_w,), jnp.int32),
            pltpu.VMEM((per_w,), jnp.float32),
            pltpu.VMEM_SHARED((n * R,), jnp.float32),
        ],
    )
    def count_kernel(t_hbm, r_hbm, out_hbm, tv, rv, segv, onesv, cnt_sh):
        c = lax.axis_index("c")
        s = lax.axis_index("s")
        wid = c * NS + s
        base = wid * per_w

        def zfill(i, _):
            onesv[pl.ds(i * N_LANES, N_LANES)] = jnp.zeros((N_LANES,), jnp.float32)
            return 0

        lax.fori_loop(0, per_w // N_LANES, zfill, 0)
        pltpu.sync_copy(onesv.at[pl.ds(0, per_s)], cnt_sh.at[pl.ds(s * per_s, per_s)])

        pltpu.sync_copy(t_hbm.at[pl.ds(base, per_w)], tv)
        pltpu.sync_copy(r_hbm.at[pl.ds(base, per_w)], rv)

        def seg_body(i, _):
            sl = pl.ds(i * N_LANES, N_LANES)
            segv[sl] = tv[sl] * R + rv[sl]
            onesv[sl] = jnp.ones((N_LANES,), jnp.float32)
            return 0

        lax.fori_loop(0, per_w // N_LANES, seg_body, 0)

        plsc.subcore_barrier()
        pltpu.sync_copy(onesv, cnt_sh.at[segv], add=True)
        plsc.subcore_barrier()

        pltpu.sync_copy(cnt_sh.at[pl.ds(s * per_s, per_s)], onesv)
        pltpu.sync_copy(onesv,
                        out_hbm.at[pl.ds(c * (n * R) + s * per_s, per_s)])

    return count_kernel


def _make_agg(E, n, R, D, NC, NS, B, CH, MEGA):
    Dh = D // NC
    per_tile = E // NS
    nmega = per_tile // MEGA
    nchunk = MEGA // CH
    nbatch = MEGA // B
    nb2 = nbatch // 2
    mesh = plsc.VectorSubcoreMesh(core_axis_name="c", subcore_axis_name="s")
    rows_a = 8 * (n // (8 * NS))
    rows_last = n - (NS - 1) * rows_a

    def _row_chunks(total):
        out, off = [], 0
        while off < total:
            sz = min(B, total - off)
            out.append((off, sz))
            off += sz
        return out

    @functools.partial(
        pl.kernel,
        mesh=mesh,
        out_type=jax.ShapeDtypeStruct((NC * n, Dh), jnp.float32),
        compiler_params=pltpu.CompilerParams(use_tc_tiling_on_sc=False),
        scratch_types=[
            pltpu.VMEM((CH,), jnp.int32),
            pltpu.VMEM((CH,), jnp.int32),
            pltpu.VMEM((MEGA,), jnp.int32),
            pltpu.VMEM((MEGA,), jnp.int32),
            pltpu.VMEM((CH,), jnp.int32),
            pltpu.VMEM((MEGA,), jnp.float32),
            pltpu.VMEM((B,), jnp.int32),
            pltpu.VMEM((B,), jnp.int32),
            pltpu.VMEM((B,), jnp.int32),
            pltpu.VMEM((B,), jnp.int32),
            pltpu.VMEM((B, Dh), jnp.float32),
            pltpu.VMEM((B, Dh), jnp.float32),
            pltpu.VMEM_SHARED((n, Dh), jnp.float32),
            pltpu.SemaphoreType.DMA,
            pltpu.SemaphoreType.DMA,
        ],
    )
    def agg_kernel(h_hbm, r_hbm, t_hbm, z_hbm, winv_hbm, out_hbm,
                   hv, rv, tv, gidxv, segv, wv, gb0, tb0, gb1, tb1,
                   zbuf0, zbuf1, agg_sh, sem_g0, sem_g1):
        c = lax.axis_index("c")
        s = lax.axis_index("s")

        def zero_body(e, _):
            for k in range(Dh // N_LANES):
                zbuf0[e, pl.ds(k * N_LANES, N_LANES)] = jnp.zeros(
                    (N_LANES,), jnp.float32)
            return 0

        lax.fori_loop(0, B, zero_body, 0)

        @pl.when(s < NS - 1)
        def _():
            for off, sz in _row_chunks(rows_a):
                pltpu.sync_copy(zbuf0.at[pl.ds(0, sz), :],
                                agg_sh.at[pl.ds(s * rows_a + off, sz), :])

        @pl.when(s == NS - 1)
        def _():
            for off, sz in _row_chunks(rows_last):
                pltpu.sync_copy(
                    zbuf0.at[pl.ds(0, sz), :],
                    agg_sh.at[pl.ds((NS - 1) * rows_a + off, sz), :])

        plsc.subcore_barrier()

        def fill_idx(q, gb, tb):
            def cp(j, _):
                sl = pl.ds(j * N_LANES, N_LANES)
                src = pl.ds(q * B + j * N_LANES, N_LANES)
                gb[sl] = gidxv[src]
                tb[sl] = tv[src]
                return 0

            lax.fori_loop(0, B // N_LANES, cp, 0)

        def scale(zbuf, q):
            def group_body(g, _):
                w16 = wv[pl.ds(q * B + g * N_LANES, N_LANES)]
                for l in range(N_LANES):
                    wspl = _lane_splat(w16, l)
                    row = g * N_LANES + l
                    for k in range(Dh // N_LANES):
                        sl = pl.ds(k * N_LANES, N_LANES)
                        zbuf[row, sl] = zbuf[row, sl] * wspl
                return 0

            lax.fori_loop(0, B // N_LANES, group_body, 0, unroll=2)

        def mega_body(m, _):
            tbase = s * per_tile + m * MEGA

            pltpu.sync_copy(t_hbm.at[pl.ds(tbase, MEGA)], tv)

            def chunk_body(mc, _):
                off = mc * CH
                pltpu.sync_copy(h_hbm.at[pl.ds(tbase + off, CH)], hv)
                pltpu.sync_copy(r_hbm.at[pl.ds(tbase + off, CH)], rv)

                def idx_body(i, _):
                    sl = pl.ds(i * N_LANES, N_LANES)
                    dst = pl.ds(off + i * N_LANES, N_LANES)
                    r16 = rv[sl]
                    gidxv[dst] = (r16 * n + hv[sl]) * NC + c
                    segv[sl] = tv[dst] * R + r16
                    return 0

                lax.fori_loop(0, CH // N_LANES, idx_body, 0)
                pltpu.sync_copy(winv_hbm.at[segv], wv.at[pl.ds(off, CH)])
                return 0

            lax.fori_loop(0, nchunk, chunk_body, 0)

            fill_idx(0, gb0, tb0)
            pltpu.async_copy(z_hbm.at[gb0], zbuf0, sem_g0)

            def pair_body(p, _):
                pltpu.make_async_copy(z_hbm.at[gb0], zbuf0, sem_g0).wait()
                fill_idx(2 * p + 1, gb1, tb1)
                pltpu.async_copy(z_hbm.at[gb1], zbuf1, sem_g1)
                scale(zbuf0, 2 * p)
                pltpu.sync_copy(zbuf0, agg_sh.at[tb0], add=True)

                pltpu.make_async_copy(z_hbm.at[gb1], zbuf1, sem_g1).wait()
                fill_idx(2 * p + 2, gb0, tb0)
                pltpu.async_copy(z_hbm.at[gb0], zbuf0, sem_g0)
                scale(zbuf1, 2 * p + 1)
                pltpu.sync_copy(zbuf1, agg_sh.at[tb1], add=True)
                return 0

            lax.fori_loop(0, nb2, pair_body, 0)
            pltpu.make_async_copy(z_hbm.at[gb0], zbuf0, sem_g0).wait()
            scale(zbuf0, nbatch - 1)
            pltpu.sync_copy(zbuf0, agg_sh.at[tb0], add=True)
            return 0

        lax.fori_loop(0, nmega, mega_body, 0)

        plsc.subcore_barrier()

        @pl.when(s < NS - 1)
        def _():
            for off, sz in _row_chunks(rows_a):
                row0 = s * rows_a + off
                pltpu.sync_copy(agg_sh.at[pl.ds(row0, sz), :],
                                zbuf0.at[pl.ds(0, sz), :])
                pltpu.sync_copy(zbuf0.at[pl.ds(0, sz), :],
                                out_hbm.at[pl.ds(c * n + row0, sz), :])

        @pl.when(s == NS - 1)
        def _():
            for off, sz in _row_chunks(rows_last):
                row0 = (NS - 1) * rows_a + off
                pltpu.sync_copy(agg_sh.at[pl.ds(row0, sz), :],
                                zbuf0.at[pl.ds(0, sz), :])
                pltpu.sync_copy(zbuf0.at[pl.ds(0, sz), :],
                                out_hbm.at[pl.ds(c * n + row0, sz), :])

    return agg_kernel


def _winv_body(cnt_ref, winv_ref):
    c = cnt_ref[0] + cnt_ref[1]
    winv_ref[...] = 1.0 / jnp.maximum(c, 1.0)


def _winv(cnt, n, R):
    rows = (n * R) // 128
    cnt3 = cnt.reshape(2, rows, 128)
    out = pl.pallas_call(
        _winv_body,
        out_shape=jax.ShapeDtypeStruct((rows, 128), jnp.float32),
    )(cnt3)
    return out.reshape(n * R)


def _zself_body(x_ref, w_ref, root_ref, b_ref, z_ref, self_ref):
    r = pl.program_id(1)
    x = x_ref[...]
    z_ref[0] = jnp.dot(x, w_ref[0], preferred_element_type=jnp.float32)

    @pl.when(r == 0)
    def _():
        self_ref[...] = (
            jnp.dot(x, root_ref[...], preferred_element_type=jnp.float32)
            + b_ref[...]
        )


def _zself(x, W, root, b, NB):
    n, d_in = x.shape
    R, _, d_out = W.shape
    ni = n // NB
    z, self_ = pl.pallas_call(
        _zself_body,
        grid=(ni, R),
        in_specs=[
            pl.BlockSpec((NB, d_in), lambda i, r: (i, 0)),
            pl.BlockSpec((1, d_in, d_out), lambda i, r: (r, 0, 0)),
            pl.BlockSpec((d_in, d_out), lambda i, r: (0, 0)),
            pl.BlockSpec((1, d_out), lambda i, r: (0, 0)),
        ],
        out_specs=[
            pl.BlockSpec((1, NB, d_out), lambda i, r: (r, i, 0)),
            pl.BlockSpec((NB, d_out), lambda i, r: (i, 0)),
        ],
        out_shape=[
            jax.ShapeDtypeStruct((R, n, d_out), jnp.float32),
            jax.ShapeDtypeStruct((n, d_out), jnp.float32),
        ],
    )(x, W, root, b.reshape(1, d_out))
    return z.reshape(R * n, d_out), self_


def _zself_fused_body(agg_ref, s1_ref, w_ref, root_ref, b_ref,
                      z_ref, self_ref, x_s):
    r = pl.program_id(1)

    @pl.when(r == 0)
    def _():
        v = jnp.concatenate([agg_ref[0], agg_ref[1]], axis=1) + s1_ref[...]
        x_s[...] = jnp.maximum(v, 0.0)

    x = x_s[...]
    z_ref[0] = jnp.dot(x, w_ref[0], preferred_element_type=jnp.float32)

    @pl.when(r == 0)
    def _():
        self_ref[...] = (
            jnp.dot(x, root_ref[...], preferred_element_type=jnp.float32)
            + b_ref[...]
        )


def _zself_fused(agg, self1, W, root, b, NB):
    _, n, dh = agg.shape
    R, d_in, d_out = W.shape
    ni = n // NB
    z, self_ = pl.pallas_call(
        _zself_fused_body,
        grid=(ni, R),
        in_specs=[
            pl.BlockSpec((2, NB, dh), lambda i, r: (0, i, 0)),
            pl.BlockSpec((NB, 2 * dh), lambda i, r: (i, 0)),
            pl.BlockSpec((1, d_in, d_out), lambda i, r: (r, 0, 0)),
            pl.BlockSpec((d_in, d_out), lambda i, r: (0, 0)),
            pl.BlockSpec((1, d_out), lambda i, r: (0, 0)),
        ],
        out_specs=[
            pl.BlockSpec((1, NB, d_out), lambda i, r: (r, i, 0)),
            pl.BlockSpec((NB, d_out), lambda i, r: (i, 0)),
        ],
        out_shape=[
            jax.ShapeDtypeStruct((R, n, d_out), jnp.float32),
            jax.ShapeDtypeStruct((n, d_out), jnp.float32),
        ],
        scratch_shapes=[pltpu.VMEM((NB, d_in), jnp.float32)],
    )(agg, self1, W, root, b.reshape(1, d_out))
    return z.reshape(R * n, d_out), self_


def _combine_body(agg_ref, self_ref, out_ref, *, relu):
    v = jnp.concatenate([agg_ref[0], agg_ref[1]], axis=1) + self_ref[...]
    out_ref[...] = jnp.maximum(v, 0.0) if relu else v


def _combine(agg, self_, NB, relu):
    _, n, dh = agg.shape
    d = 2 * dh
    ni = n // NB
    return pl.pallas_call(
        functools.partial(_combine_body, relu=relu),
        grid=(ni,),
        in_specs=[
            pl.BlockSpec((2, NB, dh), lambda i: (0, i, 0)),
            pl.BlockSpec((NB, d), lambda i: (i, 0)),
        ],
        out_specs=pl.BlockSpec((NB, d), lambda i: (i, 0)),
        out_shape=jax.ShapeDtypeStruct((n, d), jnp.float32),
    )(agg, self_)


def _final_body(agg_ref, self_ref, out_ref, acc_ref, *, ni, n):
    i = pl.program_id(0)

    @pl.when(i == 0)
    def _():
        acc_ref[...] = jnp.zeros_like(acc_ref)

    x = jnp.concatenate([agg_ref[0], agg_ref[1]], axis=1) + self_ref[...]
    nb, d = x.shape
    acc_ref[...] += jnp.sum(x.reshape(nb // 8, 8, d), axis=0)

    @pl.when(i == ni - 1)
    def _():
        out_ref[...] = jnp.sum(acc_ref[...], axis=0, keepdims=True) * (1.0 / n)


def _final(agg, self_, NB):
    _, n, dh = agg.shape
    d = 2 * dh
    ni = n // NB
    return pl.pallas_call(
        functools.partial(_final_body, ni=ni, n=n),
        grid=(ni,),
        in_specs=[
            pl.BlockSpec((2, NB, dh), lambda i: (0, i, 0)),
            pl.BlockSpec((NB, d), lambda i: (i, 0)),
        ],
        out_specs=pl.BlockSpec((1, d), lambda i: (0, 0)),
        out_shape=jax.ShapeDtypeStruct((1, d), jnp.float32),
        scratch_shapes=[pltpu.VMEM((8, d), jnp.float32)],
    )(agg, self_)


def kernel(h, r, t, emb, W1, root1, b1, W2, root2, b2):
    E = h.shape[0]
    n, d_in = emb.shape
    R = W1.shape[0]
    D = W1.shape[2]

    info = plsc.get_sparse_core_info()
    NC, NS = info.num_cores, info.num_subcores

    NB = 1000
    B = 400
    CH = 2000
    MEGA = 10000
    Dh = D // NC

    count_k = _make_count(E, n, R, NC, NS)
    agg_k = _make_agg(E, n, R, D, NC, NS, B, CH, MEGA)

    cnt = count_k(t, r).reshape(NC, n * R)
    winv = _winv(cnt, n, R)

    z1, self1 = _zself(emb, W1, root1, b1, NB)
    agg1 = agg_k(h, r, t, z1.reshape(R * n * NC, Dh), winv).reshape(NC, n, Dh)

    z2, self2 = _zself_fused(agg1, self1, W2, root2, b2, NB)
    agg2 = agg_k(h, r, t, z2.reshape(R * n * NC, Dh), winv).reshape(NC, n, Dh)
    return _final(agg2, self2, NB)
